# Initial kernel scaffold; baseline (speedup 1.0000x reference)
import functools

import jax
import jax.numpy as jnp
from jax import lax
from jax.experimental import pallas as pl
from jax.experimental.pallas import tpu as pltpu
from jax.experimental.pallas import tpu_sc as plsc


def _swish(x):
    return x * jax.nn.sigmoid(x)


def _prep_tables(v, u, batchc, W1a, W1b, W1d, Wn1a, Wn1c, be1r, bn1r):
    n, d = v.shape
    g = u.shape[0]
    R = 1000
    grid = n // R

    def body(v_ref, u_ref, b_ref, w1a, w1b, w1d, wn1a, wn1c, be1_, bn1_,
             a2_ref, bt_ref, vn0_ref):
        oh = (b_ref[...] == lax.broadcasted_iota(jnp.int32, (1, g), 1)
              ).astype(jnp.float32)
        cu = jnp.dot(u_ref[...], w1d[...], preferred_element_type=jnp.float32)
        cn = jnp.dot(u_ref[...], wn1c[...], preferred_element_type=jnp.float32)
        vb = v_ref[...]
        a2_ref[...] = (jnp.dot(vb, w1a[...], preferred_element_type=jnp.float32)
                       + jnp.dot(oh, cu, preferred_element_type=jnp.float32)
                       + be1_[...])
        bt_ref[...] = jnp.dot(vb, w1b[...], preferred_element_type=jnp.float32)
        vn0_ref[...] = (jnp.dot(vb, wn1a[...], preferred_element_type=jnp.float32)
                        + jnp.dot(oh, cn, preferred_element_type=jnp.float32)
                        + bn1_[...])

    wspec = pl.BlockSpec((128, 128), lambda i: (0, 0))
    bspec = pl.BlockSpec((1, 128), lambda i: (0, 0))
    return pl.pallas_call(
        body,
        grid=(grid,),
        in_specs=[
            pl.BlockSpec((R, d), lambda i: (i, 0)),
            pl.BlockSpec((g, d), lambda i: (0, 0)),
            pl.BlockSpec((R, 1), lambda i: (i, 0)),
            wspec, wspec, wspec, wspec, wspec, bspec, bspec,
        ],
        out_specs=[
            pl.BlockSpec((R, 128), lambda i: (i, 0)),
            pl.BlockSpec((R, 128), lambda i: (i, 0)),
            pl.BlockSpec((R, 128), lambda i: (i, 0)),
        ],
        out_shape=[
            jax.ShapeDtypeStruct((n, 128), jnp.float32),
            jax.ShapeDtypeStruct((n, 128), jnp.float32),
            jax.ShapeDtypeStruct((n, 128), jnp.float32),
        ],
    )(v, u, batchc, W1a, W1b, W1d, Wn1a, Wn1c, be1r, bn1r)


def _gather_pairsum(A2, Bt, row3, col3):
    n, d = A2.shape
    nchunks = row3.shape[0]
    e_total = nchunks * 128
    ntiles = 32
    per_tile = nchunks // ntiles
    rem = nchunks - per_tile * ntiles

    mesh = plsc.VectorSubcoreMesh(core_axis_name="core",
                                  subcore_axis_name="subcore")

    @functools.partial(
        pl.kernel,
        out_type=jax.ShapeDtypeStruct((e_total, d), jnp.float32),
        mesh=mesh,
        scratch_types=[
            pltpu.VMEM((128, d), jnp.float32),
            pltpu.VMEM((128, d), jnp.float32),
            pltpu.VMEM((1, 128), jnp.int32),
            pltpu.VMEM((1, 128), jnp.int32),
        ],
    )
    def k(a_hbm, b_hbm, row_hbm, col_hbm, o_hbm, a_v, b_v, ridx, cidx):
        cid = lax.axis_index("core")
        sid = lax.axis_index("subcore")
        wid = sid * 2 + cid

        def do_chunk(ch):
            pltpu.sync_copy(row_hbm.at[ch], ridx)
            pltpu.sync_copy(col_hbm.at[ch], cidx)
            pltpu.sync_copy(a_hbm.at[ridx.at[0]], a_v)
            pltpu.sync_copy(b_hbm.at[cidx.at[0]], b_v)

            @pl.loop(0, 128)
            def _(r):
                for c in range(0, d, 16):
                    s = (pl.ds(r, 1), pl.ds(c, 16))
                    a_v.at[s][...] = a_v.at[s][...] + b_v.at[s][...]

            pltpu.sync_copy(a_v, o_hbm.at[pl.ds(ch * 128, 128)])

        @pl.loop(0, per_tile)
        def _(i):
            do_chunk(wid * per_tile + i)

        if rem:
            @pl.when(wid < rem)
            def _():
                do_chunk(ntiles * per_tile + wid)

    return k(A2, Bt, row3, col3)


def _edge_mlp(GAB, e, W1c, We2, be2r):
    m, d = e.shape
    RE = 4000
    grid = m // RE

    def body(gab_ref, e_ref, w1c, we2, be2_, o_ref):
        x = gab_ref[...] + jnp.dot(e_ref[...], w1c[...],
                                   preferred_element_type=jnp.float32)
        h = _swish(x)
        y = jnp.dot(h, we2[...], preferred_element_type=jnp.float32) + be2_[...]
        o_ref[...] = _swish(y)

    return pl.pallas_call(
        body,
        grid=(grid,),
        in_specs=[
            pl.BlockSpec((RE, 128), lambda i: (i, 0)),
            pl.BlockSpec((RE, d), lambda i: (i, 0)),
            pl.BlockSpec((d, 128), lambda i: (0, 0)),
            pl.BlockSpec((128, 128), lambda i: (0, 0)),
            pl.BlockSpec((1, 128), lambda i: (0, 0)),
        ],
        out_specs=pl.BlockSpec((RE, 128), lambda i: (i, 0)),
        out_shape=jax.ShapeDtypeStruct((m, 128), jnp.float32),
    )(GAB, e, W1c, We2, be2r)


def _scatter_partials(ep, col3, n):
    m, d = ep.shape
    nchunks = col3.shape[0]
    ntiles = 32
    per_tile = nchunks // ntiles
    rem = nchunks - per_tile * ntiles
    rows_per_sub = n // 16
    ZR = 125

    mesh = plsc.VectorSubcoreMesh(core_axis_name="core",
                                  subcore_axis_name="subcore")

    @functools.partial(
        pl.kernel,
        out_type=jax.ShapeDtypeStruct((2, n, d), jnp.float32),
        mesh=mesh,
        scratch_types=[
            pltpu.VMEM((128, d), jnp.float32),
            pltpu.VMEM((1, 128), jnp.int32),
            pltpu.VMEM((ZR, d), jnp.float32),
            pltpu.VMEM_SHARED((n, d), jnp.float32),
        ],
    )
    def k(ep_hbm, col_hbm, o_hbm, data_v, cidx, zbuf, agg_sh):
        cid = lax.axis_index("core")
        sid = lax.axis_index("subcore")
        wid = sid * 2 + cid

        @pl.loop(0, ZR)
        def _(r):
            for c in range(0, d, 16):
                zbuf.at[pl.ds(r, 1), pl.ds(c, 16)][...] = jnp.zeros(
                    (1, 16), jnp.float32)

        @pl.loop(0, rows_per_sub // ZR)
        def _(j):
            pltpu.sync_copy(zbuf,
                            agg_sh.at[pl.ds(sid * rows_per_sub + j * ZR, ZR)])

        plsc.subcore_barrier()

        def do_chunk(ch):
            pltpu.sync_copy(col_hbm.at[ch], cidx)
            pltpu.sync_copy(ep_hbm.at[pl.ds(ch * 128, 128)], data_v)
            pltpu.sync_copy(data_v, agg_sh.at[cidx.at[0]], add=True)

        @pl.loop(0, per_tile)
        def _(i):
            do_chunk(wid * per_tile + i)

        if rem:
            @pl.when(wid < rem)
            def _():
                do_chunk(ntiles * per_tile + wid)

        plsc.subcore_barrier()
        pltpu.sync_copy(agg_sh.at[pl.ds(sid * rows_per_sub, rows_per_sub)],
                        o_hbm.at[cid, pl.ds(sid * rows_per_sub, rows_per_sub)])

    return k(ep, col3)


def _node_mlp(P0, P1, Vn0, batchr, Wn1b, Wn2, bn2r, g):
    n, d = Vn0.shape
    R = 1000
    grid = n // R

    def body(p0_ref, p1_ref, vn0_ref, b_ref, wn1b, wn2, bn2_,
             vp_ref, agge_ref, aggn_ref):
        i = pl.program_id(0)
        agg = p0_ref[...] + p1_ref[...]
        x = vn0_ref[...] + jnp.dot(agg, wn1b[...],
                                   preferred_element_type=jnp.float32)
        h = _swish(x)
        vp = _swish(jnp.dot(h, wn2[...], preferred_element_type=jnp.float32)
                    + bn2_[...])
        vp_ref[...] = vp
        oht = (lax.broadcasted_iota(jnp.int32, (g, 1), 0) == b_ref[...]
               ).astype(jnp.float32)
        de = jnp.dot(oht, agg, preferred_element_type=jnp.float32)
        dn = jnp.dot(oht, vp, preferred_element_type=jnp.float32)

        @pl.when(i == 0)
        def _():
            agge_ref[...] = de
            aggn_ref[...] = dn

        @pl.when(i != 0)
        def _():
            agge_ref[...] = agge_ref[...] + de
            aggn_ref[...] = aggn_ref[...] + dn

    return pl.pallas_call(
        body,
        grid=(grid,),
        in_specs=[
            pl.BlockSpec((R, d), lambda i: (i, 0)),
            pl.BlockSpec((R, d), lambda i: (i, 0)),
            pl.BlockSpec((R, 128), lambda i: (i, 0)),
            pl.BlockSpec((1, R), lambda i: (0, i)),
            pl.BlockSpec((128, 128), lambda i: (0, 0)),
            pl.BlockSpec((128, 128), lambda i: (0, 0)),
            pl.BlockSpec((1, 128), lambda i: (0, 0)),
        ],
        out_specs=[
            pl.BlockSpec((R, 128), lambda i: (i, 0)),
            pl.BlockSpec((g, 128), lambda i: (0, 0)),
            pl.BlockSpec((g, 128), lambda i: (0, 0)),
        ],
        out_shape=[
            jax.ShapeDtypeStruct((n, 128), jnp.float32),
            jax.ShapeDtypeStruct((g, 128), jnp.float32),
            jax.ShapeDtypeStruct((g, 128), jnp.float32),
        ],
    )(P0, P1, Vn0, batchr, Wn1b, Wn2, bn2r)


def _global_mlp(u, aggE, aggN, Wg1a, Wg1b, Wg1c, bg1r, Wg2, bg2r):
    g, d = u.shape

    def body(u_ref, ae_ref, an_ref, wa, wb, wc, bg1_, wg2, bg2_, o_ref):
        x = (jnp.dot(u_ref[...], wa[...], preferred_element_type=jnp.float32)
             + jnp.dot(ae_ref[...], wb[...], preferred_element_type=jnp.float32)
             + jnp.dot(an_ref[...], wc[...], preferred_element_type=jnp.float32)
             + bg1_[...])
        h = _swish(x)
        o_ref[...] = _swish(jnp.dot(h, wg2[...],
                                    preferred_element_type=jnp.float32)
                            + bg2_[...])

    return pl.pallas_call(
        body,
        out_shape=jax.ShapeDtypeStruct((g, 128), jnp.float32),
    )(u, aggE, aggN, Wg1a, Wg1b, Wg1c, bg1r, Wg2, bg2r)


def kernel(v, e, u, edge_index, batch,
           We1, be1, We2, be2, Wn1, bn1, Wn2, bn2, Wg1, bg1, Wg2, bg2):
    n, d = v.shape
    m = e.shape[0]
    g = u.shape[0]

    row = edge_index[0]
    col = edge_index[1]
    nchunks = m // 128
    row3 = row.reshape(nchunks, 1, 128)
    col3 = col.reshape(nchunks, 1, 128)
    batchc = batch.reshape(n, 1)
    batchr = batch.reshape(1, n)

    W1a, W1b, W1c, W1d = We1[0:d], We1[d:2 * d], We1[2 * d:3 * d], We1[3 * d:]
    Wn1a, Wn1b, Wn1c = Wn1[0:d], Wn1[d:d + 128], Wn1[d + 128:]
    Wg1a, Wg1b, Wg1c = Wg1[0:d], Wg1[d:d + 128], Wg1[d + 128:]

    be1r = be1.reshape(1, -1)
    be2r = be2.reshape(1, -1)
    bn1r = bn1.reshape(1, -1)
    bn2r = bn2.reshape(1, -1)
    bg1r = bg1.reshape(1, -1)
    bg2r = bg2.reshape(1, -1)

    A2, Bt, Vn0 = _prep_tables(v, u, batchc, W1a, W1b, W1d, Wn1a, Wn1c,
                               be1r, bn1r)
    GAB = _gather_pairsum(A2, Bt, row3, col3)
    ep = _edge_mlp(GAB, e, W1c, We2, be2r)
    P = _scatter_partials(ep, col3, n)
    vp, aggE, aggN = _node_mlp(P[0], P[1], Vn0, batchr, Wn1b, Wn2, bn2r, g)
    up = _global_mlp(u, aggE, aggN, Wg1a, Wg1b, Wg1c, bg1r, Wg2, bg2r)
    return (vp, ep, up)


# speedup vs baseline: 7.0252x; 7.0252x over previous
import functools

import jax
import jax.numpy as jnp
from jax import lax
from jax.experimental import pallas as pl
from jax.experimental.pallas import tpu as pltpu
from jax.experimental.pallas import tpu_sc as plsc


def _swish(x):
    return x * jax.nn.sigmoid(x)


def _prep_tables(v, u, batchc, W1a, W1b, W1d, Wn1a, Wn1c, be1r, bn1r):
    n, d = v.shape
    g = u.shape[0]
    R = 1000
    grid = n // R

    def body(v_ref, u_ref, b_ref, w1a, w1b, w1d, wn1a, wn1c, be1_, bn1_,
             a2_ref, bt_ref, vn0_ref):
        oh = (b_ref[...] == lax.broadcasted_iota(jnp.int32, (1, g), 1)
              ).astype(jnp.float32)
        cu = jnp.dot(u_ref[...], w1d[...], preferred_element_type=jnp.float32)
        cn = jnp.dot(u_ref[...], wn1c[...], preferred_element_type=jnp.float32)
        vb = v_ref[...]
        a2_ref[...] = (jnp.dot(vb, w1a[...], preferred_element_type=jnp.float32)
                       + jnp.dot(oh, cu, preferred_element_type=jnp.float32)
                       + be1_[...])
        bt_ref[...] = jnp.dot(vb, w1b[...], preferred_element_type=jnp.float32)
        vn0_ref[...] = (jnp.dot(vb, wn1a[...], preferred_element_type=jnp.float32)
                        + jnp.dot(oh, cn, preferred_element_type=jnp.float32)
                        + bn1_[...])

    wspec = pl.BlockSpec((128, 128), lambda i: (0, 0))
    bspec = pl.BlockSpec((1, 128), lambda i: (0, 0))
    return pl.pallas_call(
        body,
        grid=(grid,),
        in_specs=[
            pl.BlockSpec((R, d), lambda i: (i, 0)),
            pl.BlockSpec((g, d), lambda i: (0, 0)),
            pl.BlockSpec((R, 1), lambda i: (i, 0)),
            wspec, wspec, wspec, wspec, wspec, bspec, bspec,
        ],
        out_specs=[
            pl.BlockSpec((R, 128), lambda i: (i, 0)),
            pl.BlockSpec((R, 128), lambda i: (i, 0)),
            pl.BlockSpec((R, 128), lambda i: (i, 0)),
        ],
        out_shape=[
            jax.ShapeDtypeStruct((n, 128), jnp.float32),
            jax.ShapeDtypeStruct((n, 128), jnp.float32),
            jax.ShapeDtypeStruct((n, 128), jnp.float32),
        ],
    )(v, u, batchc, W1a, W1b, W1d, Wn1a, Wn1c, be1r, bn1r)


def _gather_pairsum(A2, Bt, row3, col3):
    n, d = A2.shape
    nchunks = row3.shape[0]
    e_total = nchunks * 128
    ntiles = 32
    per_tile = nchunks // ntiles
    rem = nchunks - per_tile * ntiles

    mesh = plsc.VectorSubcoreMesh(core_axis_name="core",
                                  subcore_axis_name="subcore")

    @functools.partial(
        pl.kernel,
        out_type=jax.ShapeDtypeStruct((e_total, d), jnp.float32),
        mesh=mesh,
        scratch_types=[
            pltpu.VMEM((128, d), jnp.float32),
            pltpu.VMEM((128, d), jnp.float32),
            pltpu.VMEM((1, 128), jnp.int32),
            pltpu.VMEM((1, 128), jnp.int32),
        ],
    )
    def k(a_hbm, b_hbm, row_hbm, col_hbm, o_hbm, a_v, b_v, ridx, cidx):
        cid = lax.axis_index("core")
        sid = lax.axis_index("subcore")
        wid = sid * 2 + cid

        def do_chunk(ch):
            pltpu.sync_copy(row_hbm.at[ch], ridx)
            pltpu.sync_copy(col_hbm.at[ch], cidx)
            pltpu.sync_copy(a_hbm.at[ridx.at[0]], a_v)
            pltpu.sync_copy(b_hbm.at[cidx.at[0]], b_v)

            @pl.loop(0, 128)
            def _(r):
                for c in range(0, d, 16):
                    s = (pl.ds(r, 1), pl.ds(c, 16))
                    a_v.at[s][...] = a_v.at[s][...] + b_v.at[s][...]

            pltpu.sync_copy(a_v, o_hbm.at[pl.ds(ch * 128, 128)])

        @pl.loop(0, per_tile)
        def _(i):
            do_chunk(wid * per_tile + i)

        if rem:
            @pl.when(wid < rem)
            def _():
                do_chunk(ntiles * per_tile + wid)

    return k(A2, Bt, row3, col3)


def _edge_mlp(GAB, e, W1c, We2, be2r):
    m, d = e.shape
    RE = 4000
    grid = m // RE

    def body(gab_ref, e_ref, w1c, we2, be2_, o_ref):
        x = gab_ref[...] + jnp.dot(e_ref[...], w1c[...],
                                   preferred_element_type=jnp.float32)
        h = _swish(x)
        y = jnp.dot(h, we2[...], preferred_element_type=jnp.float32) + be2_[...]
        o_ref[...] = _swish(y)

    return pl.pallas_call(
        body,
        grid=(grid,),
        in_specs=[
            pl.BlockSpec((RE, 128), lambda i: (i, 0)),
            pl.BlockSpec((RE, d), lambda i: (i, 0)),
            pl.BlockSpec((d, 128), lambda i: (0, 0)),
            pl.BlockSpec((128, 128), lambda i: (0, 0)),
            pl.BlockSpec((1, 128), lambda i: (0, 0)),
        ],
        out_specs=pl.BlockSpec((RE, 128), lambda i: (i, 0)),
        out_shape=jax.ShapeDtypeStruct((m, 128), jnp.float32),
    )(GAB, e, W1c, We2, be2r)


def _scatter_partials(ep, col3, n_pad):
    m, d = ep.shape
    nchunks = col3.shape[0]
    ntiles = 32
    per_tile = nchunks // ntiles
    rem = nchunks - per_tile * ntiles
    rows_per_sub = n_pad // 16
    zfull = rows_per_sub // 128
    zrem = rows_per_sub - zfull * 128

    mesh = plsc.VectorSubcoreMesh(core_axis_name="core",
                                  subcore_axis_name="subcore")

    @functools.partial(
        pl.kernel,
        out_type=jax.ShapeDtypeStruct((2, n_pad, d), jnp.float32),
        mesh=mesh,
        scratch_types=[
            pltpu.VMEM((128, d), jnp.float32),
            pltpu.VMEM((1, 128), jnp.int32),
            pltpu.VMEM_SHARED((n_pad, d), jnp.float32),
        ],
    )
    def k(ep_hbm, col_hbm, o_hbm, data_v, cidx, agg_sh):
        cid = lax.axis_index("core")
        sid = lax.axis_index("subcore")
        wid = sid * 2 + cid

        @pl.loop(0, 128)
        def _(r):
            for c in range(0, d, 16):
                data_v.at[pl.ds(r, 1), pl.ds(c, 16)][...] = jnp.zeros(
                    (1, 16), jnp.float32)

        @pl.loop(0, zfull)
        def _(j):
            pltpu.sync_copy(data_v,
                            agg_sh.at[pl.ds(sid * rows_per_sub + j * 128, 128)])

        if zrem:
            pltpu.sync_copy(
                data_v.at[pl.ds(0, zrem)],
                agg_sh.at[pl.ds(sid * rows_per_sub + zfull * 128, zrem)])

        plsc.subcore_barrier()

        def do_chunk(ch):
            pltpu.sync_copy(col_hbm.at[ch], cidx)
            pltpu.sync_copy(ep_hbm.at[pl.ds(ch * 128, 128)], data_v)
            pltpu.sync_copy(data_v, agg_sh.at[cidx.at[0]], add=True)

        @pl.loop(0, per_tile)
        def _(i):
            do_chunk(wid * per_tile + i)

        if rem:
            @pl.when(wid < rem)
            def _():
                do_chunk(ntiles * per_tile + wid)

        plsc.subcore_barrier()
        pltpu.sync_copy(agg_sh.at[pl.ds(sid * rows_per_sub, rows_per_sub)],
                        o_hbm.at[cid, pl.ds(sid * rows_per_sub, rows_per_sub)])

    return k(ep, col3)


def _node_mlp(P0, P1, Vn0, batchr, Wn1b, Wn2, bn2r, g):
    n, d = Vn0.shape
    R = 1000
    grid = n // R

    def body(p0_ref, p1_ref, vn0_ref, b_ref, wn1b, wn2, bn2_,
             vp_ref, agge_ref, aggn_ref):
        i = pl.program_id(0)
        agg = p0_ref[...] + p1_ref[...]
        x = vn0_ref[...] + jnp.dot(agg, wn1b[...],
                                   preferred_element_type=jnp.float32)
        h = _swish(x)
        vp = _swish(jnp.dot(h, wn2[...], preferred_element_type=jnp.float32)
                    + bn2_[...])
        vp_ref[...] = vp
        oht = (lax.broadcasted_iota(jnp.int32, (g, 1), 0) == b_ref[0]
               ).astype(jnp.float32)
        de = jnp.dot(oht, agg, preferred_element_type=jnp.float32)
        dn = jnp.dot(oht, vp, preferred_element_type=jnp.float32)

        @pl.when(i == 0)
        def _():
            agge_ref[...] = de
            aggn_ref[...] = dn

        @pl.when(i != 0)
        def _():
            agge_ref[...] = agge_ref[...] + de
            aggn_ref[...] = aggn_ref[...] + dn

    return pl.pallas_call(
        body,
        grid=(grid,),
        in_specs=[
            pl.BlockSpec((R, d), lambda i: (i, 0)),
            pl.BlockSpec((R, d), lambda i: (i, 0)),
            pl.BlockSpec((R, 128), lambda i: (i, 0)),
            pl.BlockSpec((1, 1, R), lambda i: (i, 0, 0)),
            pl.BlockSpec((128, 128), lambda i: (0, 0)),
            pl.BlockSpec((128, 128), lambda i: (0, 0)),
            pl.BlockSpec((1, 128), lambda i: (0, 0)),
        ],
        out_specs=[
            pl.BlockSpec((R, 128), lambda i: (i, 0)),
            pl.BlockSpec((g, 128), lambda i: (0, 0)),
            pl.BlockSpec((g, 128), lambda i: (0, 0)),
        ],
        out_shape=[
            jax.ShapeDtypeStruct((n, 128), jnp.float32),
            jax.ShapeDtypeStruct((g, 128), jnp.float32),
            jax.ShapeDtypeStruct((g, 128), jnp.float32),
        ],
    )(P0, P1, Vn0, batchr, Wn1b, Wn2, bn2r)


def _global_mlp(u, aggE, aggN, Wg1a, Wg1b, Wg1c, bg1r, Wg2, bg2r):
    g, d = u.shape

    def body(u_ref, ae_ref, an_ref, wa, wb, wc, bg1_, wg2, bg2_, o_ref):
        x = (jnp.dot(u_ref[...], wa[...], preferred_element_type=jnp.float32)
             + jnp.dot(ae_ref[...], wb[...], preferred_element_type=jnp.float32)
             + jnp.dot(an_ref[...], wc[...], preferred_element_type=jnp.float32)
             + bg1_[...])
        h = _swish(x)
        o_ref[...] = _swish(jnp.dot(h, wg2[...],
                                    preferred_element_type=jnp.float32)
                            + bg2_[...])

    return pl.pallas_call(
        body,
        out_shape=jax.ShapeDtypeStruct((g, 128), jnp.float32),
    )(u, aggE, aggN, Wg1a, Wg1b, Wg1c, bg1r, Wg2, bg2r)


def kernel(v, e, u, edge_index, batch,
           We1, be1, We2, be2, Wn1, bn1, Wn2, bn2, Wg1, bg1, Wg2, bg2):
    n, d = v.shape
    m = e.shape[0]
    g = u.shape[0]

    row = edge_index[0]
    col = edge_index[1]
    nchunks = m // 128
    row3 = row.reshape(nchunks, 1, 128)
    col3 = col.reshape(nchunks, 1, 128)
    batchc = batch.reshape(n, 1)
    batchr = batch.reshape(n // 1000, 1, 1000)

    W1a, W1b, W1c, W1d = We1[0:d], We1[d:2 * d], We1[2 * d:3 * d], We1[3 * d:]
    Wn1a, Wn1b, Wn1c = Wn1[0:d], Wn1[d:d + 128], Wn1[d + 128:]
    Wg1a, Wg1b, Wg1c = Wg1[0:d], Wg1[d:d + 128], Wg1[d + 128:]

    be1r = be1.reshape(1, -1)
    be2r = be2.reshape(1, -1)
    bn1r = bn1.reshape(1, -1)
    bn2r = bn2.reshape(1, -1)
    bg1r = bg1.reshape(1, -1)
    bg2r = bg2.reshape(1, -1)

    A2, Bt, Vn0 = _prep_tables(v, u, batchc, W1a, W1b, W1d, Wn1a, Wn1c,
                               be1r, bn1r)
    GAB = _gather_pairsum(A2, Bt, row3, col3)
    ep = _edge_mlp(GAB, e, W1c, We2, be2r)
    n_pad = ((n + 127) // 128) * 128
    P = _scatter_partials(ep, col3, n_pad)
    vp, aggE, aggN = _node_mlp(P[0], P[1], Vn0, batchr, Wn1b, Wn2, bn2r, g)
    up = _global_mlp(u, aggE, aggN, Wg1a, Wg1b, Wg1c, bg1r, Wg2, bg2r)
    return (vp, ep, up)
